# Initial kernel scaffold; baseline (speedup 1.0000x reference)
import jax
import jax.numpy as jnp
from jax.experimental import pallas as pl


def kernel(x, edge_index, batch, action, W1, b1, W2, b2, W3, b3, Wp1, bp1, Wp2, bp2, Wp3, bp3):
    raise NotImplementedError("write your pallas kernel here")


# speedup vs baseline: 39.6405x; 39.6405x over previous
import functools

import jax
import jax.numpy as jnp
from jax import lax
from jax.experimental import pallas as pl
from jax.experimental.pallas import tpu as pltpu
from jax.experimental.pallas import tpu_sc as plsc

N_SUBCORES = 16
N_CORES = 2
NW = N_CORES * N_SUBCORES
CHUNK = 128


def _round_up(a, m):
    return (a + m - 1) // m * m


@functools.partial(jax.jit, static_argnums=(2, 3))
def _sc_degree(dst_w, zeros, n_pad, nch):
    mesh = plsc.VectorSubcoreMesh(core_axis_name="c", subcore_axis_name="s")
    stripe = n_pad // N_SUBCORES

    @functools.partial(
        pl.kernel,
        mesh=mesh,
        out_type=jax.ShapeDtypeStruct((N_CORES, n_pad), jnp.float32),
        compiler_params=pltpu.CompilerParams(use_tc_tiling_on_sc=False),
        scratch_types=[
            pltpu.VMEM((nch, CHUNK), jnp.int32),
            pltpu.VMEM((CHUNK,), jnp.float32),
            pltpu.VMEM_SHARED((n_pad,), jnp.float32),
        ],
    )
    def k(dst_hbm, zeros_hbm, out_hbm, dst_v, ones_v, acc_sh):
        c = lax.axis_index("c")
        s = lax.axis_index("s")
        wid = s * N_CORES + c
        pltpu.sync_copy(dst_hbm.at[wid], dst_v)
        for i in range(CHUNK // 16):
            ones_v[pl.ds(i * 16, 16)] = jnp.ones((16,), jnp.float32)
        pltpu.sync_copy(zeros_hbm.at[pl.ds(s * stripe, stripe)],
                        acc_sh.at[pl.ds(s * stripe, stripe)])
        plsc.subcore_barrier()

        def body(j, _):
            pltpu.sync_copy(ones_v, acc_sh.at[dst_v.at[j]], add=True)
            return _
        lax.fori_loop(0, nch, body, None)
        plsc.subcore_barrier()
        pltpu.sync_copy(acc_sh.at[pl.ds(s * stripe, stripe)],
                        out_hbm.at[c, pl.ds(s * stripe, stripe)])

    return k(dst_w, zeros)


@functools.partial(jax.jit, static_argnums=(3, 4))
def _sc_edge_pass(scaled, src_w, dst_w, n_pad, nch):
    F = scaled.shape[1]
    mesh = plsc.VectorSubcoreMesh(core_axis_name="c", subcore_axis_name="s")
    stripe = n_pad // N_SUBCORES

    @functools.partial(
        pl.kernel,
        mesh=mesh,
        out_type=jax.ShapeDtypeStruct((N_CORES, n_pad, F), jnp.float32),
        compiler_params=pltpu.CompilerParams(use_tc_tiling_on_sc=False),
        scratch_types=[
            pltpu.VMEM((nch, CHUNK), jnp.int32),
            pltpu.VMEM((nch, CHUNK), jnp.int32),
            pltpu.VMEM((CHUNK, F), jnp.float32),
            pltpu.VMEM((CHUNK, F), jnp.float32),
            pltpu.VMEM_SHARED((n_pad, F), jnp.float32),
            pltpu.SemaphoreType.DMA,
            pltpu.SemaphoreType.DMA,
        ],
    )
    def k(scaled_hbm, src_hbm, dst_hbm, out_hbm,
          src_v, dst_v, rows0_v, rows1_v, acc_sh, sem0, sem1):
        c = lax.axis_index("c")
        s = lax.axis_index("s")
        wid = s * N_CORES + c
        pltpu.sync_copy(src_hbm.at[wid], src_v)
        pltpu.sync_copy(dst_hbm.at[wid], dst_v)
        pltpu.sync_copy(scaled_hbm.at[pl.ds(s * stripe, stripe)],
                        acc_sh.at[pl.ds(s * stripe, stripe)])
        plsc.subcore_barrier()

        rows = (rows0_v, rows1_v)
        sems = (sem0, sem1)
        pltpu.async_copy(scaled_hbm.at[src_v.at[0]], rows0_v, sem0)

        def body(jo, _):
            for b in range(2):
                j = jo * 2 + b

                @pl.when(j + 1 < nch)
                def _prefetch():
                    pltpu.async_copy(scaled_hbm.at[src_v.at[j + 1]],
                                     rows[(b + 1) % 2], sems[(b + 1) % 2])

                pltpu.make_async_copy(scaled_hbm.at[src_v.at[j]],
                                      rows[b], sems[b]).wait()
                pltpu.sync_copy(rows[b], acc_sh.at[dst_v.at[j]], add=True)
            return _
        lax.fori_loop(0, nch // 2, body, None)
        plsc.subcore_barrier()
        pltpu.sync_copy(acc_sh.at[pl.ds(s * stripe, stripe)],
                        out_hbm.at[c, pl.ds(s * stripe, stripe), :])

    return k(scaled, src_w, dst_w)


def _tc_first(deg_pT, x_pad, W1):
    n_pad = x_pad.shape[0]

    def k(degp_ref, x_ref, w_ref, out_scaled, out_dinv):
        deg = degp_ref[:, 0:1] + degp_ref[:, 1:2] + 1.0
        dinv2 = lax.rsqrt(jnp.maximum(deg, 1.0))
        h = jnp.dot(x_ref[...], w_ref[...], preferred_element_type=jnp.float32)
        out_scaled[...] = h * dinv2
        out_dinv[...] = dinv2

    return pl.pallas_call(
        k,
        out_shape=[
            jax.ShapeDtypeStruct((n_pad, W1.shape[1]), jnp.float32),
            jax.ShapeDtypeStruct((n_pad, 1), jnp.float32),
        ],
    )(deg_pT, x_pad, W1)


def _tc_mid(p, scaled_prev, dinv, b, W, relu=True):
    n_pad = scaled_prev.shape[0]

    def k(p_ref, sp_ref, dinv_ref, b_ref, w_ref, out_ref):
        combine = p_ref[0] + p_ref[1] - sp_ref[...]
        a = combine * dinv_ref[...] + b_ref[...]
        if relu:
            a = jnp.maximum(a, 0.0)
        h = jnp.dot(a, w_ref[...], preferred_element_type=jnp.float32)
        out_ref[...] = h * dinv_ref[...]

    return pl.pallas_call(
        k,
        out_shape=jax.ShapeDtypeStruct((n_pad, W.shape[1]), jnp.float32),
    )(p, scaled_prev, dinv, b, W)


def _tc_final(p, scaled3, dinv, b3, batch2, ones_col, action,
              Wp1, bp1, Wp2a, Wp2b, bp2, Wp3, bp3, n_graphs):
    def k(p_ref, sp_ref, dinv_ref, b3_ref, batch_ref, ones_ref, act_ref,
          wp1_ref, bp1_ref, wp2a_ref, wp2b_ref, bp2_ref, wp3_ref, bp3_ref,
          out_ref):
        n_pad = sp_ref.shape[0]
        combine = p_ref[0] + p_ref[1] - sp_ref[...]
        h3 = combine * dinv_ref[...] + b3_ref[...]
        gids = lax.broadcasted_iota(jnp.int32, (n_pad, n_graphs), 1)
        onehot = (batch_ref[...] == gids).astype(jnp.float32)
        cdims = (((0,), (0,)), ((), ()))
        sums = lax.dot_general(onehot, h3, cdims,
                               preferred_element_type=jnp.float32)
        cnt = lax.dot_general(onehot, ones_ref[...], cdims,
                              preferred_element_type=jnp.float32)
        pooled = sums / jnp.maximum(cnt, 1.0)
        fp = jnp.maximum(
            jnp.dot(pooled, wp1_ref[...], preferred_element_type=jnp.float32)
            + bp1_ref[...], 0.0)
        pol = (jnp.dot(fp, wp2a_ref[...], preferred_element_type=jnp.float32)
               + jnp.dot(act_ref[...], wp2b_ref[...],
                         preferred_element_type=jnp.float32)
               + bp2_ref[...])
        pol = jnp.maximum(pol, 0.0)
        out_ref[...] = (jnp.dot(pol, wp3_ref[...],
                                preferred_element_type=jnp.float32)
                        + bp3_ref[...])

    return pl.pallas_call(
        k,
        out_shape=jax.ShapeDtypeStruct((n_graphs, 1), jnp.float32),
    )(p, scaled3, dinv, b3, batch2, ones_col, action,
      Wp1, bp1, Wp2a, Wp2b, bp2, Wp3, bp3)


def kernel(x, edge_index, batch, action,
           W1, b1, W2, b2, W3, b3, Wp1, bp1, Wp2, bp2, Wp3, bp3):
    n, _ = x.shape
    g = action.shape[0]
    e = edge_index.shape[1]

    n_pad = _round_up(n + N_SUBCORES, N_SUBCORES * 8 * 8)
    nch = _round_up(-(-e // NW), 2 * CHUNK) // CHUNK
    e_pad = NW * nch * CHUNK

    src = edge_index[0].astype(jnp.int32)
    dst = edge_index[1].astype(jnp.int32)
    npad_e = e_pad - e
    pad_ar = jnp.arange(npad_e, dtype=jnp.int32)
    src_w = jnp.concatenate([src, pad_ar % n]).reshape(NW, nch, CHUNK)
    dst_w = jnp.concatenate([dst, n + (pad_ar % N_SUBCORES)]).reshape(NW, nch, CHUNK)

    x_pad = jnp.pad(x, ((0, n_pad - n), (0, 0)))
    batch2 = jnp.pad(batch.astype(jnp.int32), (0, n_pad - n),
                     constant_values=g)[:, None]
    zeros = jnp.zeros((n_pad,), jnp.float32)
    ones_col = jnp.ones((n_pad, 1), jnp.float32)

    deg_p = _sc_degree(dst_w, zeros, n_pad, nch)

    scaled1, dinv = _tc_first(deg_p.T, x_pad, W1)
    p1 = _sc_edge_pass(scaled1, src_w, dst_w, n_pad, nch)
    scaled2 = _tc_mid(p1, scaled1, dinv, b1[None, :], W2, relu=True)
    p2 = _sc_edge_pass(scaled2, src_w, dst_w, n_pad, nch)
    scaled3 = _tc_mid(p2, scaled2, dinv, b2[None, :], W3, relu=True)
    p3 = _sc_edge_pass(scaled3, src_w, dst_w, n_pad, nch)
    return _tc_final(p3, scaled3, dinv, b3[None, :], batch2, ones_col,
                     action, Wp1, bp1[None, :], Wp2[:64], Wp2[64:],
                     bp2[None, :], Wp3, bp3[None, :], g)


# speedup vs baseline: 46.2963x; 1.1679x over previous
import functools

import jax
import jax.numpy as jnp
from jax import lax
from jax.experimental import pallas as pl
from jax.experimental.pallas import tpu as pltpu
from jax.experimental.pallas import tpu_sc as plsc

N_SUBCORES = 16
N_CORES = 2
NW = N_CORES * N_SUBCORES
CHUNK = 128


def _round_up(a, m):
    return (a + m - 1) // m * m


@functools.partial(jax.jit, static_argnums=(2, 3))
def _sc_degree(dst_w, zeros, n_pad, nch):
    mesh = plsc.VectorSubcoreMesh(core_axis_name="c", subcore_axis_name="s")
    stripe = n_pad // N_SUBCORES

    @functools.partial(
        pl.kernel,
        mesh=mesh,
        out_type=jax.ShapeDtypeStruct((N_CORES, n_pad), jnp.float32),
        compiler_params=pltpu.CompilerParams(use_tc_tiling_on_sc=False),
        scratch_types=[
            pltpu.VMEM((nch, CHUNK), jnp.int32),
            pltpu.VMEM((CHUNK,), jnp.float32),
            pltpu.VMEM_SHARED((n_pad,), jnp.float32),
            pltpu.SemaphoreType.DMA,
        ],
    )
    def k(dst_hbm, zeros_hbm, out_hbm, dst_v, ones_v, acc_sh, sem):
        c = lax.axis_index("c")
        s = lax.axis_index("s")
        wid = s * N_CORES + c
        pltpu.sync_copy(dst_hbm.at[wid], dst_v)
        for i in range(CHUNK // 16):
            ones_v[pl.ds(i * 16, 16)] = jnp.ones((16,), jnp.float32)
        pltpu.sync_copy(zeros_hbm.at[pl.ds(s * stripe, stripe)],
                        acc_sh.at[pl.ds(s * stripe, stripe)])
        plsc.subcore_barrier()

        def body(j, _):
            pltpu.async_copy(ones_v, acc_sh.at[dst_v.at[j]], sem, add=True)
            return _
        lax.fori_loop(0, nch, body, None)

        def drain(j, _):
            pltpu.make_async_copy(ones_v, acc_sh.at[dst_v.at[j]], sem).wait()
            return _
        lax.fori_loop(0, nch, drain, None)
        plsc.subcore_barrier()
        pltpu.sync_copy(acc_sh.at[pl.ds(s * stripe, stripe)],
                        out_hbm.at[c, pl.ds(s * stripe, stripe)])

    return k(dst_w, zeros)


@functools.partial(jax.jit, static_argnums=(3, 4))
def _sc_edge_pass(scaled, src_w, dst_w, n_pad, nch):
    F = scaled.shape[1]
    mesh = plsc.VectorSubcoreMesh(core_axis_name="c", subcore_axis_name="s")
    stripe = n_pad // N_SUBCORES

    @functools.partial(
        pl.kernel,
        mesh=mesh,
        out_type=jax.ShapeDtypeStruct((N_CORES, n_pad, F), jnp.float32),
        compiler_params=pltpu.CompilerParams(use_tc_tiling_on_sc=False),
        scratch_types=[
            pltpu.VMEM((nch, CHUNK), jnp.int32),
            pltpu.VMEM((nch, CHUNK), jnp.int32),
            pltpu.VMEM((4, CHUNK, F), jnp.float32),
            pltpu.VMEM_SHARED((n_pad, F), jnp.float32),
            [pltpu.SemaphoreType.DMA] * 4,
            [pltpu.SemaphoreType.DMA] * 4,
        ],
    )
    def k(scaled_hbm, src_hbm, dst_hbm, out_hbm,
          src_v, dst_v, rows_v, acc_sh, gsems, ssems):
        c = lax.axis_index("c")
        s = lax.axis_index("s")
        wid = s * N_CORES + c
        pltpu.sync_copy(src_hbm.at[wid], src_v)
        pltpu.sync_copy(dst_hbm.at[wid], dst_v)
        pltpu.sync_copy(scaled_hbm.at[pl.ds(s * stripe, stripe)],
                        acc_sh.at[pl.ds(s * stripe, stripe)])
        plsc.subcore_barrier()

        def gath(j, b):
            return pltpu.make_async_copy(
                scaled_hbm.at[src_v.at[j]], rows_v.at[b], gsems[b])

        def scat(j, b):
            return pltpu.make_async_copy(
                rows_v.at[b], acc_sh.at[dst_v.at[j]], ssems[b])

        gath(0, 0).start()
        gath(1, 1).start()

        def body(jo, _):
            for b4 in range(4):
                j = jo * 4 + b4
                nb = (b4 + 2) % 4

                @pl.when(j >= 2)
                def _free():
                    scat(j - 2, nb).wait()

                @pl.when(j + 2 < nch)
                def _prefetch():
                    gath(j + 2, nb).start()

                gath(j, b4).wait()
                scat(j, b4).start(add=True)
            return _
        lax.fori_loop(0, nch // 4, body, None)
        scat(nch - 2, (nch - 2) % 4).wait()
        scat(nch - 1, (nch - 1) % 4).wait()
        plsc.subcore_barrier()
        pltpu.sync_copy(acc_sh.at[pl.ds(s * stripe, stripe)],
                        out_hbm.at[c, pl.ds(s * stripe, stripe), :])

    return k(scaled, src_w, dst_w)


def _tc_matmul(x_pad, W1):
    n_pad = x_pad.shape[0]

    def k(x_ref, w_ref, out_ref):
        out_ref[...] = jnp.dot(x_ref[...], w_ref[...],
                               preferred_element_type=jnp.float32)

    return pl.pallas_call(
        k,
        out_shape=jax.ShapeDtypeStruct((n_pad, W1.shape[1]), jnp.float32),
    )(x_pad, W1)


def _tc_first(deg_pT, h1):
    n_pad = h1.shape[0]

    def k(degp_ref, h_ref, out_scaled, out_dinv):
        deg = degp_ref[:, 0:1] + degp_ref[:, 1:2] + 1.0
        dinv2 = lax.rsqrt(jnp.maximum(deg, 1.0))
        out_scaled[...] = h_ref[...] * dinv2
        out_dinv[...] = dinv2

    return pl.pallas_call(
        k,
        out_shape=[
            jax.ShapeDtypeStruct(h1.shape, jnp.float32),
            jax.ShapeDtypeStruct((n_pad, 1), jnp.float32),
        ],
    )(deg_pT, h1)


def _tc_mid(p, scaled_prev, dinv, b, W, relu=True):
    n_pad = scaled_prev.shape[0]

    def k(p_ref, sp_ref, dinv_ref, b_ref, w_ref, out_ref):
        combine = p_ref[0] + p_ref[1] - sp_ref[...]
        a = combine * dinv_ref[...] + b_ref[...]
        if relu:
            a = jnp.maximum(a, 0.0)
        h = jnp.dot(a, w_ref[...], preferred_element_type=jnp.float32)
        out_ref[...] = h * dinv_ref[...]

    return pl.pallas_call(
        k,
        out_shape=jax.ShapeDtypeStruct((n_pad, W.shape[1]), jnp.float32),
    )(p, scaled_prev, dinv, b, W)


def _tc_final(p, scaled3, dinv, b3, batch2, ones_col, action,
              Wp1, bp1, Wp2a, Wp2b, bp2, Wp3, bp3, n_graphs):
    def k(p_ref, sp_ref, dinv_ref, b3_ref, batch_ref, ones_ref, act_ref,
          wp1_ref, bp1_ref, wp2a_ref, wp2b_ref, bp2_ref, wp3_ref, bp3_ref,
          out_ref):
        n_pad = sp_ref.shape[0]
        combine = p_ref[0] + p_ref[1] - sp_ref[...]
        h3 = combine * dinv_ref[...] + b3_ref[...]
        gids = lax.broadcasted_iota(jnp.int32, (n_pad, n_graphs), 1)
        onehot = (batch_ref[...] == gids).astype(jnp.float32)
        cdims = (((0,), (0,)), ((), ()))
        sums = lax.dot_general(onehot, h3, cdims,
                               preferred_element_type=jnp.float32)
        cnt = lax.dot_general(onehot, ones_ref[...], cdims,
                              preferred_element_type=jnp.float32)
        pooled = sums / jnp.maximum(cnt, 1.0)
        fp = jnp.maximum(
            jnp.dot(pooled, wp1_ref[...], preferred_element_type=jnp.float32)
            + bp1_ref[...], 0.0)
        pol = (jnp.dot(fp, wp2a_ref[...], preferred_element_type=jnp.float32)
               + jnp.dot(act_ref[...], wp2b_ref[...],
                         preferred_element_type=jnp.float32)
               + bp2_ref[...])
        pol = jnp.maximum(pol, 0.0)
        out_ref[...] = (jnp.dot(pol, wp3_ref[...],
                                preferred_element_type=jnp.float32)
                        + bp3_ref[...])

    return pl.pallas_call(
        k,
        out_shape=jax.ShapeDtypeStruct((n_graphs, 1), jnp.float32),
    )(p, scaled3, dinv, b3, batch2, ones_col, action,
      Wp1, bp1, Wp2a, Wp2b, bp2, Wp3, bp3)


def kernel(x, edge_index, batch, action,
           W1, b1, W2, b2, W3, b3, Wp1, bp1, Wp2, bp2, Wp3, bp3):
    n, _ = x.shape
    g = action.shape[0]
    e = edge_index.shape[1]

    n_pad = _round_up(n + N_SUBCORES, N_SUBCORES * 8 * 8)
    nch = _round_up(-(-e // NW), 2 * CHUNK) // CHUNK
    e_pad = NW * nch * CHUNK

    src = edge_index[0].astype(jnp.int32)
    dst = edge_index[1].astype(jnp.int32)
    npad_e = e_pad - e
    pad_ar = jnp.arange(npad_e, dtype=jnp.int32)
    src_w = jnp.concatenate([src, pad_ar % n]).reshape(NW, nch, CHUNK)
    dst_w = jnp.concatenate([dst, n + (pad_ar % N_SUBCORES)]).reshape(NW, nch, CHUNK)

    x_pad = jnp.pad(x, ((0, n_pad - n), (0, 0)))
    batch2 = jnp.pad(batch.astype(jnp.int32), (0, n_pad - n),
                     constant_values=g)[:, None]
    zeros = jnp.zeros((n_pad,), jnp.float32)
    ones_col = jnp.ones((n_pad, 1), jnp.float32)

    deg_p = _sc_degree(dst_w, zeros, n_pad, nch)

    h1 = _tc_matmul(x_pad, W1)
    scaled1, dinv = _tc_first(deg_p.T, h1)
    p1 = _sc_edge_pass(scaled1, src_w, dst_w, n_pad, nch)
    scaled2 = _tc_mid(p1, scaled1, dinv, b1[None, :], W2, relu=True)
    p2 = _sc_edge_pass(scaled2, src_w, dst_w, n_pad, nch)
    scaled3 = _tc_mid(p2, scaled2, dinv, b2[None, :], W3, relu=True)
    p3 = _sc_edge_pass(scaled3, src_w, dst_w, n_pad, nch)
    return _tc_final(p3, scaled3, dinv, b3[None, :], batch2, ones_col,
                     action, Wp1, bp1[None, :], Wp2[:64], Wp2[64:],
                     bp2[None, :], Wp3, bp3[None, :], g)


# speedup vs baseline: 49.1959x; 1.0626x over previous
import functools

import jax
import jax.numpy as jnp
from jax import lax
from jax.experimental import pallas as pl
from jax.experimental.pallas import tpu as pltpu
from jax.experimental.pallas import tpu_sc as plsc

N_SUBCORES = 16
N_CORES = 2
NW = N_CORES * N_SUBCORES
CHUNK = 128


def _round_up(a, m):
    return (a + m - 1) // m * m


def _stage_indices(ei_hbm, src_v, dst_v, wid, base_rows, rows_total, rem, n):
    base = wid * base_rows
    pltpu.sync_copy(ei_hbm.at[0, pl.ds(base, base_rows)],
                    src_v.at[pl.ds(0, base_rows)])
    pltpu.sync_copy(ei_hbm.at[1, pl.ds(base, base_rows)],
                    dst_v.at[pl.ds(0, base_rows)])

    if rem:
        @pl.when(wid < rem)
        def _extra():
            pltpu.sync_copy(ei_hbm.at[0, rows_total - rem + wid],
                            src_v.at[base_rows])
            pltpu.sync_copy(ei_hbm.at[1, rows_total - rem + wid],
                            dst_v.at[base_rows])

    @pl.when(wid >= rem)
    def _pad():
        for i in range(CHUNK // 16):
            iota = lax.iota(jnp.int32, 16)
            src_v[base_rows, pl.ds(i * 16, 16)] = iota + (i * 16)
            dst_v[base_rows, pl.ds(i * 16, 16)] = iota + n


@functools.partial(jax.jit, static_argnums=(2, 3, 4, 5))
def _sc_degree(ei3, zeros, n_pad, base_rows, rem, n):
    mesh = plsc.VectorSubcoreMesh(core_axis_name="c", subcore_axis_name="s")
    stripe = n_pad // N_SUBCORES
    rows_total = ei3.shape[1]
    nch = base_rows + 1

    @functools.partial(
        pl.kernel,
        mesh=mesh,
        out_type=[jax.ShapeDtypeStruct((n_pad,), jnp.float32)] * N_CORES,
        compiler_params=pltpu.CompilerParams(use_tc_tiling_on_sc=False),
        scratch_types=[
            pltpu.VMEM((nch, CHUNK), jnp.int32),
            pltpu.VMEM((nch, CHUNK), jnp.int32),
            pltpu.VMEM((CHUNK,), jnp.float32),
            pltpu.VMEM_SHARED((n_pad,), jnp.float32),
            pltpu.SemaphoreType.DMA,
        ],
    )
    def k(ei_hbm, zeros_hbm, out0_hbm, out1_hbm,
          src_v, dst_v, ones_v, acc_sh, sem):
        c = lax.axis_index("c")
        s = lax.axis_index("s")
        wid = s * N_CORES + c
        _stage_indices(ei_hbm, src_v, dst_v, wid, base_rows, rows_total,
                       rem, n)
        for i in range(CHUNK // 16):
            ones_v[pl.ds(i * 16, 16)] = jnp.ones((16,), jnp.float32)
        pltpu.sync_copy(zeros_hbm.at[pl.ds(s * stripe, stripe)],
                        acc_sh.at[pl.ds(s * stripe, stripe)])
        plsc.subcore_barrier()

        def body(j, _):
            pltpu.async_copy(ones_v, acc_sh.at[dst_v.at[j]], sem, add=True)
            return _
        lax.fori_loop(0, nch, body, None)

        def drain(j, _):
            pltpu.make_async_copy(ones_v, acc_sh.at[dst_v.at[j]], sem).wait()
            return _
        lax.fori_loop(0, nch, drain, None)
        plsc.subcore_barrier()

        @pl.when(c == 0)
        def _w0():
            pltpu.sync_copy(acc_sh.at[pl.ds(s * stripe, stripe)],
                            out0_hbm.at[pl.ds(s * stripe, stripe)])

        @pl.when(c == 1)
        def _w1():
            pltpu.sync_copy(acc_sh.at[pl.ds(s * stripe, stripe)],
                            out1_hbm.at[pl.ds(s * stripe, stripe)])

    return k(ei3, zeros)


@functools.partial(jax.jit, static_argnums=(2, 3, 4, 5))
def _sc_edge_pass(scaled, ei3, n_pad, base_rows, rem, n):
    F = scaled.shape[1]
    mesh = plsc.VectorSubcoreMesh(core_axis_name="c", subcore_axis_name="s")
    stripe = n_pad // N_SUBCORES
    rows_total = ei3.shape[1]
    nch = base_rows + 1
    main = max(((nch - 3) // 4) * 4, 0)
    stage = F <= 32

    @functools.partial(
        pl.kernel,
        mesh=mesh,
        out_type=jax.ShapeDtypeStruct((N_CORES, n_pad, F), jnp.float32),
        compiler_params=pltpu.CompilerParams(use_tc_tiling_on_sc=False),
        scratch_types=[
            pltpu.VMEM((nch, CHUNK), jnp.int32),
            pltpu.VMEM((nch, CHUNK), jnp.int32),
            pltpu.VMEM((4, CHUNK, F), jnp.float32),
            pltpu.VMEM_SHARED((n_pad, F), jnp.float32),
            pltpu.VMEM_SHARED((n_pad, F) if stage else (8, F), jnp.float32),
            [pltpu.SemaphoreType.DMA] * 4,
            [pltpu.SemaphoreType.DMA] * 4,
        ],
    )
    def k(scaled_hbm, ei_hbm, out_hbm,
          src_v, dst_v, rows_v, acc_sh, sca_sh, gsems, ssems):
        c = lax.axis_index("c")
        s = lax.axis_index("s")
        wid = s * N_CORES + c
        _stage_indices(ei_hbm, src_v, dst_v, wid, base_rows, rows_total,
                       rem, n)
        pltpu.sync_copy(scaled_hbm.at[pl.ds(s * stripe, stripe)],
                        acc_sh.at[pl.ds(s * stripe, stripe)])
        if stage:
            pltpu.sync_copy(scaled_hbm.at[pl.ds(s * stripe, stripe)],
                            sca_sh.at[pl.ds(s * stripe, stripe)])
        plsc.subcore_barrier()

        gsrc = sca_sh if stage else scaled_hbm

        def gath(j, b):
            return pltpu.make_async_copy(
                gsrc.at[src_v.at[j]], rows_v.at[b], gsems[b])

        def scat(j, b):
            return pltpu.make_async_copy(
                rows_v.at[b], acc_sh.at[dst_v.at[j]], ssems[b])

        def step(j, j4, first, last):
            nb = (j4 + 2) % 4
            if first:
                @pl.when(j >= 2)
                def _free():
                    scat(j - 2, nb).wait()
            elif j >= 2:
                scat(j - 2, nb).wait()
            if not last:
                gath(j + 2, nb).start()
            gath(j, j4).wait()
            scat(j, j4).start(add=True)

        gath(0, 0).start()
        gath(1, 1).start()

        def body(jo, _):
            for b4 in range(4):
                step(jo * 4 + b4, b4, first=True, last=False)
            return _
        lax.fori_loop(0, main // 4, body, None)
        for j in range(main, nch):
            step(j, j % 4, first=False, last=(j + 2 >= nch))
        scat(nch - 2, (nch - 2) % 4).wait()
        scat(nch - 1, (nch - 1) % 4).wait()
        plsc.subcore_barrier()
        pltpu.sync_copy(acc_sh.at[pl.ds(s * stripe, stripe)],
                        out_hbm.at[c, pl.ds(s * stripe, stripe), :])

    return k(scaled, ei3)


def _tc_matmul(x_pad, W1):
    n_pad = x_pad.shape[0]

    def k(x_ref, w_ref, out_ref):
        out_ref[...] = jnp.dot(x_ref[...], w_ref[...],
                               preferred_element_type=jnp.float32)

    return pl.pallas_call(
        k,
        out_shape=jax.ShapeDtypeStruct((n_pad, W1.shape[1]), jnp.float32),
    )(x_pad, W1)


def _tc_first(deg0, deg1, h1):
    n_pad = h1.shape[0]

    def k(d0_ref, d1_ref, h_ref, out_scaled, out_dinv):
        deg = d0_ref[...] + d1_ref[...] + 1.0
        dinv2 = lax.rsqrt(jnp.maximum(deg, 1.0))
        out_scaled[...] = h_ref[...] * dinv2
        out_dinv[...] = dinv2

    return pl.pallas_call(
        k,
        out_shape=[
            jax.ShapeDtypeStruct(h1.shape, jnp.float32),
            jax.ShapeDtypeStruct((n_pad, 1), jnp.float32),
        ],
    )(deg0, deg1, h1)


def _tc_mid(p, scaled_prev, dinv, b, W, relu=True):
    n_pad = scaled_prev.shape[0]

    def k(p_ref, sp_ref, dinv_ref, b_ref, w_ref, out_ref):
        combine = p_ref[0] + p_ref[1] - sp_ref[...]
        a = combine * dinv_ref[...] + b_ref[...]
        if relu:
            a = jnp.maximum(a, 0.0)
        h = jnp.dot(a, w_ref[...], preferred_element_type=jnp.float32)
        out_ref[...] = h * dinv_ref[...]

    return pl.pallas_call(
        k,
        out_shape=jax.ShapeDtypeStruct((n_pad, W.shape[1]), jnp.float32),
    )(p, scaled_prev, dinv, b, W)


def _tc_final(p, scaled3, dinv, b3, batch2, ones_col, action,
              Wp1, bp1, Wp2a, Wp2b, bp2, Wp3, bp3, n_graphs):
    def k(p_ref, sp_ref, dinv_ref, b3_ref, batch_ref, ones_ref, act_ref,
          wp1_ref, bp1_ref, wp2a_ref, wp2b_ref, bp2_ref, wp3_ref, bp3_ref,
          out_ref):
        n_pad = sp_ref.shape[0]
        combine = p_ref[0] + p_ref[1] - sp_ref[...]
        h3 = combine * dinv_ref[...] + b3_ref[...]
        gids = lax.broadcasted_iota(jnp.int32, (n_pad, n_graphs), 1)
        onehot = (batch_ref[...] == gids).astype(jnp.float32)
        cdims = (((0,), (0,)), ((), ()))
        sums = lax.dot_general(onehot, h3, cdims,
                               preferred_element_type=jnp.float32)
        cnt = lax.dot_general(onehot, ones_ref[...], cdims,
                              preferred_element_type=jnp.float32)
        pooled = sums / jnp.maximum(cnt, 1.0)
        fp = jnp.maximum(
            jnp.dot(pooled, wp1_ref[...], preferred_element_type=jnp.float32)
            + bp1_ref[...], 0.0)
        pol = (jnp.dot(fp, wp2a_ref[...], preferred_element_type=jnp.float32)
               + jnp.dot(act_ref[...], wp2b_ref[...],
                         preferred_element_type=jnp.float32)
               + bp2_ref[...])
        pol = jnp.maximum(pol, 0.0)
        out_ref[...] = (jnp.dot(pol, wp3_ref[...],
                                preferred_element_type=jnp.float32)
                        + bp3_ref[...])

    return pl.pallas_call(
        k,
        out_shape=jax.ShapeDtypeStruct((n_graphs, 1), jnp.float32),
    )(p, scaled3, dinv, b3, batch2, ones_col, action,
      Wp1, bp1, Wp2a, Wp2b, bp2, Wp3, bp3)


def kernel(x, edge_index, batch, action,
           W1, b1, W2, b2, W3, b3, Wp1, bp1, Wp2, bp2, Wp3, bp3):
    n, _ = x.shape
    g = action.shape[0]
    e = edge_index.shape[1]
    assert e % CHUNK == 0
    rows_total = e // CHUNK
    base_rows = rows_total // NW
    rem = rows_total % NW

    n_pad = _round_up(n + N_SUBCORES, N_SUBCORES * 8 * 8)
    ei3 = edge_index.astype(jnp.int32).reshape(2, rows_total, CHUNK)

    x_pad = jnp.pad(x, ((0, n_pad - n), (0, 0)))
    batch2 = jnp.pad(batch.astype(jnp.int32), (0, n_pad - n),
                     constant_values=g)[:, None]
    zeros = jnp.zeros((n_pad,), jnp.float32)
    ones_col = jnp.ones((n_pad, 1), jnp.float32)

    deg0, deg1 = _sc_degree(ei3, zeros, n_pad, base_rows, rem, n)
    deg0, deg1 = deg0[:, None], deg1[:, None]

    h1 = _tc_matmul(x_pad, W1)
    scaled1, dinv = _tc_first(deg0, deg1, h1)
    p1 = _sc_edge_pass(scaled1, ei3, n_pad, base_rows, rem, n)
    scaled2 = _tc_mid(p1, scaled1, dinv, b1[None, :], W2, relu=True)
    p2 = _sc_edge_pass(scaled2, ei3, n_pad, base_rows, rem, n)
    scaled3 = _tc_mid(p2, scaled2, dinv, b2[None, :], W3, relu=True)
    p3 = _sc_edge_pass(scaled3, ei3, n_pad, base_rows, rem, n)
    return _tc_final(p3, scaled3, dinv, b3[None, :], batch2, ones_col,
                     action, Wp1, bp1[None, :], Wp2[:64], Wp2[64:],
                     bp2[None, :], Wp3, bp3[None, :], g)


# speedup vs baseline: 54.3284x; 1.1043x over previous
import functools

import jax
import jax.numpy as jnp
from jax import lax
from jax.experimental import pallas as pl
from jax.experimental.pallas import tpu as pltpu
from jax.experimental.pallas import tpu_sc as plsc

N_SUBCORES = 16
N_CORES = 2
NW = N_CORES * N_SUBCORES
CHUNK = 128


def _round_up(a, m):
    return (a + m - 1) // m * m


def _stage_indices(ei_hbm, src_v, dst_v, wid, base_rows, rows_total, rem, n):
    base = wid * base_rows
    pltpu.sync_copy(ei_hbm.at[0, pl.ds(base, base_rows)],
                    src_v.at[pl.ds(0, base_rows)])
    pltpu.sync_copy(ei_hbm.at[1, pl.ds(base, base_rows)],
                    dst_v.at[pl.ds(0, base_rows)])

    if rem:
        @pl.when(wid < rem)
        def _extra():
            pltpu.sync_copy(ei_hbm.at[0, rows_total - rem + wid],
                            src_v.at[base_rows])
            pltpu.sync_copy(ei_hbm.at[1, rows_total - rem + wid],
                            dst_v.at[base_rows])

    @pl.when(wid >= rem)
    def _pad():
        for i in range(CHUNK // 16):
            iota = lax.iota(jnp.int32, 16)
            src_v[base_rows, pl.ds(i * 16, 16)] = iota + (i * 16)
            dst_v[base_rows, pl.ds(i * 16, 16)] = iota + n


@functools.partial(jax.jit, static_argnums=(2, 3, 4, 5))
def _sc_degree(ei3, zeros, n_pad, base_rows, rem, n):
    mesh = plsc.VectorSubcoreMesh(core_axis_name="c", subcore_axis_name="s")
    stripe = n_pad // N_SUBCORES
    rows_total = ei3.shape[1]
    nch = base_rows + 1

    @functools.partial(
        pl.kernel,
        mesh=mesh,
        out_type=[jax.ShapeDtypeStruct((n_pad,), jnp.float32)] * N_CORES,
        compiler_params=pltpu.CompilerParams(use_tc_tiling_on_sc=False),
        scratch_types=[
            pltpu.VMEM((nch, CHUNK), jnp.int32),
            pltpu.VMEM((nch, CHUNK), jnp.int32),
            pltpu.VMEM((CHUNK,), jnp.float32),
            pltpu.VMEM_SHARED((n_pad,), jnp.float32),
            pltpu.SemaphoreType.DMA,
        ],
    )
    def k(ei_hbm, zeros_hbm, out0_hbm, out1_hbm,
          src_v, dst_v, ones_v, acc_sh, sem):
        c = lax.axis_index("c")
        s = lax.axis_index("s")
        wid = s * N_CORES + c
        _stage_indices(ei_hbm, src_v, dst_v, wid, base_rows, rows_total,
                       rem, n)
        for i in range(CHUNK // 16):
            ones_v[pl.ds(i * 16, 16)] = jnp.ones((16,), jnp.float32)
        pltpu.sync_copy(zeros_hbm.at[pl.ds(s * stripe, stripe)],
                        acc_sh.at[pl.ds(s * stripe, stripe)])
        plsc.subcore_barrier()

        def body(j, _):
            pltpu.async_copy(ones_v, acc_sh.at[dst_v.at[j]], sem, add=True)
            return _
        lax.fori_loop(0, nch, body, None)

        def drain(j, _):
            pltpu.make_async_copy(ones_v, acc_sh.at[dst_v.at[j]], sem).wait()
            return _
        lax.fori_loop(0, nch, drain, None)
        plsc.subcore_barrier()

        @pl.when(c == 0)
        def _w0():
            pltpu.sync_copy(acc_sh.at[pl.ds(s * stripe, stripe)],
                            out0_hbm.at[pl.ds(s * stripe, stripe)])

        @pl.when(c == 1)
        def _w1():
            pltpu.sync_copy(acc_sh.at[pl.ds(s * stripe, stripe)],
                            out1_hbm.at[pl.ds(s * stripe, stripe)])

    return k(ei3, zeros)


@functools.partial(jax.jit, static_argnums=(2, 3, 4, 5))
def _sc_edge_pass(scaled, ei3, n_pad, base_rows, rem, n):
    F = scaled.shape[1]
    mesh = plsc.VectorSubcoreMesh(core_axis_name="c", subcore_axis_name="s")
    stripe = n_pad // N_SUBCORES
    rows_total = ei3.shape[1]
    nch = base_rows + 1
    main = max(((nch - 3) // 4) * 4, 0)
    stage = F <= 32

    @functools.partial(
        pl.kernel,
        mesh=mesh,
        out_type=jax.ShapeDtypeStruct((N_CORES, n_pad, F), jnp.float32),
        compiler_params=pltpu.CompilerParams(use_tc_tiling_on_sc=False),
        scratch_types=[
            pltpu.VMEM((nch, CHUNK), jnp.int32),
            pltpu.VMEM((nch, CHUNK), jnp.int32),
            pltpu.VMEM((4, CHUNK, F), jnp.float32),
            pltpu.VMEM_SHARED((n_pad, F), jnp.float32),
            pltpu.VMEM_SHARED((n_pad, F) if stage else (8, F), jnp.float32),
            [pltpu.SemaphoreType.DMA] * 4,
            [pltpu.SemaphoreType.DMA] * 4,
        ],
    )
    def k(scaled_hbm, ei_hbm, out_hbm,
          src_v, dst_v, rows_v, acc_sh, sca_sh, gsems, ssems):
        c = lax.axis_index("c")
        s = lax.axis_index("s")
        wid = s * N_CORES + c
        _stage_indices(ei_hbm, src_v, dst_v, wid, base_rows, rows_total,
                       rem, n)
        pltpu.sync_copy(scaled_hbm.at[pl.ds(s * stripe, stripe)],
                        acc_sh.at[pl.ds(s * stripe, stripe)])
        if stage:
            pltpu.sync_copy(scaled_hbm.at[pl.ds(s * stripe, stripe)],
                            sca_sh.at[pl.ds(s * stripe, stripe)])
        plsc.subcore_barrier()

        gsrc = sca_sh if stage else scaled_hbm

        def gath(j, b):
            return pltpu.make_async_copy(
                gsrc.at[src_v.at[j]], rows_v.at[b], gsems[b])

        def scat(j, b):
            return pltpu.make_async_copy(
                rows_v.at[b], acc_sh.at[dst_v.at[j]], ssems[b])

        def step(j, j4, first, last):
            nb = (j4 + 2) % 4
            if first:
                @pl.when(j >= 2)
                def _free():
                    scat(j - 2, nb).wait()
            elif j >= 2:
                scat(j - 2, nb).wait()
            if not last:
                gath(j + 2, nb).start()
            gath(j, j4).wait()
            scat(j, j4).start(add=True)

        gath(0, 0).start()
        gath(1, 1).start()

        def body(jo, _):
            for b4 in range(4):
                step(jo * 4 + b4, b4, first=True, last=False)
            return _
        lax.fori_loop(0, main // 4, body, None)
        for j in range(main, nch):
            step(j, j % 4, first=False, last=(j + 2 >= nch))
        scat(nch - 2, (nch - 2) % 4).wait()
        scat(nch - 1, (nch - 1) % 4).wait()
        plsc.subcore_barrier()
        pltpu.sync_copy(acc_sh.at[pl.ds(s * stripe, stripe)],
                        out_hbm.at[c, pl.ds(s * stripe, stripe), :])

    return k(scaled, ei3)


def _tc_matmul(x, W1, n_pad):
    n, F = x.shape[0], W1.shape[1]

    def k(x_ref, w_ref, out_ref):
        h = jnp.dot(x_ref[...], w_ref[...], preferred_element_type=jnp.float32)
        out_ref[0:n, :] = h
        out_ref[n:n_pad, :] = jnp.zeros((n_pad - n, F), jnp.float32)

    return pl.pallas_call(
        k,
        out_shape=jax.ShapeDtypeStruct((n_pad, F), jnp.float32),
    )(x, W1)


def _tc_first(deg0, deg1, h1):
    n_pad = h1.shape[0]

    def k(d0_ref, d1_ref, h_ref, out_scaled, out_dinv):
        deg = d0_ref[...] + d1_ref[...] + 1.0
        dv = lax.rsqrt(jnp.maximum(deg, 1.0))
        out_dinv[...] = dv
        out_scaled[...] = h_ref[...] * dv[:, None]

    return pl.pallas_call(
        k,
        out_shape=[
            jax.ShapeDtypeStruct(h1.shape, jnp.float32),
            jax.ShapeDtypeStruct((n_pad,), jnp.float32),
        ],
    )(deg0, deg1, h1)


def _tc_mid(p, sp, dinv, b, W):
    n_pad = sp.shape[0]
    Fo = W.shape[1]

    def k(p_ref, sp_ref, dinv_ref, b_ref, w_ref, out_ref):
        dcol = dinv_ref[...][:, None]
        combine = p_ref[0] + p_ref[1] - sp_ref[...]
        a = jnp.maximum(combine * dcol + b_ref[...], 0.0)
        h = jnp.dot(a, w_ref[...], preferred_element_type=jnp.float32)
        out_ref[...] = h * dcol

    return pl.pallas_call(
        k,
        out_shape=jax.ShapeDtypeStruct((n_pad, Fo), jnp.float32),
    )(p, sp, dinv, b, W)


def _tc_final(p, sp, dinv, b3, batch1, action,
              Wp1, bp1, Wp2a, Wp2b, bp2, Wp3, bp3, n_graphs):
    n_pad = sp.shape[0]

    def k(p_ref, sp_ref, dinv_ref, b3_ref, batch_ref, act_ref,
          wp1_ref, bp1_ref, wp2a_ref, wp2b_ref, bp2_ref, wp3_ref, bp3_ref,
          out_ref):
        dcol = dinv_ref[...][:, None]
        combine = p_ref[0] + p_ref[1] - sp_ref[...]
        h3 = combine * dcol + b3_ref[...]
        bcol = batch_ref[...][:, None]
        gids = lax.broadcasted_iota(jnp.int32, (n_pad, n_graphs), 1)
        onehot = (bcol == gids).astype(jnp.float32)
        cdims = (((0,), (0,)), ((), ()))
        h3x = jnp.concatenate(
            [h3, jnp.ones((n_pad, 1), jnp.float32)], axis=1)
        sumsx = lax.dot_general(onehot, h3x, cdims,
                                preferred_element_type=jnp.float32)
        sums = sumsx[:, 0:h3.shape[1]]
        cnt = sumsx[:, h3.shape[1]:]
        pooled = sums / jnp.maximum(cnt, 1.0)
        fp = jnp.maximum(
            jnp.dot(pooled, wp1_ref[...], preferred_element_type=jnp.float32)
            + bp1_ref[...], 0.0)
        pol = (jnp.dot(fp, wp2a_ref[...], preferred_element_type=jnp.float32)
               + jnp.dot(act_ref[...], wp2b_ref[...],
                         preferred_element_type=jnp.float32)
               + bp2_ref[...])
        pol = jnp.maximum(pol, 0.0)
        out_ref[...] = (jnp.dot(pol, wp3_ref[...],
                                preferred_element_type=jnp.float32)
                        + bp3_ref[...])

    return pl.pallas_call(
        k,
        out_shape=jax.ShapeDtypeStruct((n_graphs, 1), jnp.float32),
    )(p, sp, dinv, b3, batch1, action,
      Wp1, bp1, Wp2a, Wp2b, bp2, Wp3, bp3)


def kernel(x, edge_index, batch, action,
           W1, b1, W2, b2, W3, b3, Wp1, bp1, Wp2, bp2, Wp3, bp3):
    n, _ = x.shape
    g = action.shape[0]
    e = edge_index.shape[1]
    assert e % CHUNK == 0
    rows_total = e // CHUNK
    base_rows = rows_total // NW
    rem = rows_total % NW

    n_pad = _round_up(n + N_SUBCORES, N_SUBCORES * 8 * 8)
    ei3 = edge_index.astype(jnp.int32).reshape(2, rows_total, CHUNK)

    batch1 = jnp.pad(batch.astype(jnp.int32), (0, n_pad - n),
                     constant_values=g)
    zeros = jnp.zeros((n_pad,), jnp.float32)

    deg0, deg1 = _sc_degree(ei3, zeros, n_pad, base_rows, rem, n)

    h1 = _tc_matmul(x, W1, n_pad)
    scaled1, dinv = _tc_first(deg0, deg1, h1)
    p1 = _sc_edge_pass(scaled1, ei3, n_pad, base_rows, rem, n)
    scaled2 = _tc_mid(p1, scaled1, dinv, b1[None, :], W2)
    p2 = _sc_edge_pass(scaled2, ei3, n_pad, base_rows, rem, n)
    scaled3 = _tc_mid(p2, scaled2, dinv, b2[None, :], W3)
    p3 = _sc_edge_pass(scaled3, ei3, n_pad, base_rows, rem, n)
    return _tc_final(p3, scaled3, dinv, b3[None, :], batch1, action,
                     Wp1, bp1[None, :], Wp2[:64], Wp2[64:],
                     bp2[None, :], Wp3, bp3[None, :], g)


# speedup vs baseline: 59.3944x; 1.0932x over previous
import functools

import jax
import jax.numpy as jnp
from jax import lax
from jax.experimental import pallas as pl
from jax.experimental.pallas import tpu as pltpu
from jax.experimental.pallas import tpu_sc as plsc

N_SUBCORES = 16
N_CORES = 2
NW = N_CORES * N_SUBCORES
CHUNK = 128


def _round_up(a, m):
    return (a + m - 1) // m * m


def _stage_indices(ei_hbm, src_v, dst_v, wid, base_rows, rows_total, rem, n):
    base = wid * base_rows
    pltpu.sync_copy(ei_hbm.at[0, pl.ds(base, base_rows)],
                    src_v.at[pl.ds(0, base_rows)])
    pltpu.sync_copy(ei_hbm.at[1, pl.ds(base, base_rows)],
                    dst_v.at[pl.ds(0, base_rows)])

    if rem:
        @pl.when(wid < rem)
        def _extra():
            pltpu.sync_copy(ei_hbm.at[0, rows_total - rem + wid],
                            src_v.at[base_rows])
            pltpu.sync_copy(ei_hbm.at[1, rows_total - rem + wid],
                            dst_v.at[base_rows])

    @pl.when(wid >= rem)
    def _pad():
        for i in range(CHUNK // 16):
            iota = lax.iota(jnp.int32, 16)
            src_v[base_rows, pl.ds(i * 16, 16)] = iota + (i * 16)
            dst_v[base_rows, pl.ds(i * 16, 16)] = iota + n


@functools.partial(jax.jit, static_argnums=(2, 3, 4, 5))
def _sc_degree(ei3, zeros, n_pad, base_rows, rem, n):
    mesh = plsc.VectorSubcoreMesh(core_axis_name="c", subcore_axis_name="s")
    stripe = n_pad // N_SUBCORES
    rows_total = ei3.shape[1]
    nch = base_rows + 1

    @functools.partial(
        pl.kernel,
        mesh=mesh,
        out_type=[jax.ShapeDtypeStruct((n_pad,), jnp.float32)] * N_CORES,
        compiler_params=pltpu.CompilerParams(use_tc_tiling_on_sc=False),
        scratch_types=[
            pltpu.VMEM((nch, CHUNK), jnp.int32),
            pltpu.VMEM((nch, CHUNK), jnp.int32),
            pltpu.VMEM((CHUNK,), jnp.float32),
            pltpu.VMEM_SHARED((n_pad,), jnp.float32),
            pltpu.SemaphoreType.DMA,
        ],
    )
    def k(ei_hbm, zeros_hbm, out0_hbm, out1_hbm,
          src_v, dst_v, ones_v, acc_sh, sem):
        c = lax.axis_index("c")
        s = lax.axis_index("s")
        wid = s * N_CORES + c
        _stage_indices(ei_hbm, src_v, dst_v, wid, base_rows, rows_total,
                       rem, n)
        for i in range(CHUNK // 16):
            ones_v[pl.ds(i * 16, 16)] = jnp.ones((16,), jnp.float32)
        pltpu.sync_copy(zeros_hbm.at[pl.ds(s * stripe, stripe)],
                        acc_sh.at[pl.ds(s * stripe, stripe)])
        plsc.subcore_barrier()

        def body(j, _):
            pltpu.async_copy(ones_v, acc_sh.at[dst_v.at[j]], sem, add=True)
            return _
        lax.fori_loop(0, nch, body, None)

        def drain(j, _):
            pltpu.make_async_copy(ones_v, acc_sh.at[dst_v.at[j]], sem).wait()
            return _
        lax.fori_loop(0, nch, drain, None)
        plsc.subcore_barrier()

        @pl.when(c == 0)
        def _w0():
            pltpu.sync_copy(acc_sh.at[pl.ds(s * stripe, stripe)],
                            out0_hbm.at[pl.ds(s * stripe, stripe)])

        @pl.when(c == 1)
        def _w1():
            pltpu.sync_copy(acc_sh.at[pl.ds(s * stripe, stripe)],
                            out1_hbm.at[pl.ds(s * stripe, stripe)])

    return k(ei3, zeros)


@functools.partial(jax.jit, static_argnums=(2, 3, 4, 5))
def _sc_edge_pass(scaled, ei3, n_pad, base_rows, rem, n):
    F = scaled.shape[1]
    mesh = plsc.VectorSubcoreMesh(core_axis_name="c", subcore_axis_name="s")
    stripe = n_pad // N_SUBCORES
    rows_total = ei3.shape[1]
    nch = base_rows + 1
    main = max(((nch - 3) // 4) * 4, 0)
    stage = F <= 32

    @functools.partial(
        pl.kernel,
        mesh=mesh,
        out_type=jax.ShapeDtypeStruct((N_CORES, n_pad, 128), jnp.float32),
        compiler_params=pltpu.CompilerParams(use_tc_tiling_on_sc=False),
        scratch_types=[
            pltpu.VMEM((nch, CHUNK), jnp.int32),
            pltpu.VMEM((nch, CHUNK), jnp.int32),
            pltpu.VMEM((4, CHUNK, F), jnp.float32),
            pltpu.VMEM_SHARED((n_pad, F), jnp.float32),
            pltpu.VMEM_SHARED((n_pad, F) if stage else (8, F), jnp.float32),
            [pltpu.SemaphoreType.DMA] * 4,
            [pltpu.SemaphoreType.DMA] * 4,
        ],
    )
    def k(scaled_hbm, ei_hbm, out_hbm,
          src_v, dst_v, rows_v, acc_sh, sca_sh, gsems, ssems):
        c = lax.axis_index("c")
        s = lax.axis_index("s")
        wid = s * N_CORES + c
        _stage_indices(ei_hbm, src_v, dst_v, wid, base_rows, rows_total,
                       rem, n)
        pltpu.sync_copy(scaled_hbm.at[pl.ds(s * stripe, stripe)],
                        acc_sh.at[pl.ds(s * stripe, stripe)])
        if stage:
            pltpu.sync_copy(scaled_hbm.at[pl.ds(s * stripe, stripe)],
                            sca_sh.at[pl.ds(s * stripe, stripe)])
        plsc.subcore_barrier()

        gsrc = sca_sh if stage else scaled_hbm

        def gath(j, b):
            return pltpu.make_async_copy(
                gsrc.at[src_v.at[j]], rows_v.at[b], gsems[b])

        def scat(j, b):
            return pltpu.make_async_copy(
                rows_v.at[b], acc_sh.at[dst_v.at[j]], ssems[b])

        def step(j, j4, first, last):
            nb = (j4 + 2) % 4
            if first:
                @pl.when(j >= 2)
                def _free():
                    scat(j - 2, nb).wait()
            elif j >= 2:
                scat(j - 2, nb).wait()
            if not last:
                gath(j + 2, nb).start()
            gath(j, j4).wait()
            scat(j, j4).start(add=True)

        gath(0, 0).start()
        gath(1, 1).start()

        def body(jo, _):
            for b4 in range(4):
                step(jo * 4 + b4, b4, first=True, last=False)
            return _
        lax.fori_loop(0, main // 4, body, None)
        for j in range(main, nch):
            step(j, j % 4, first=False, last=(j + 2 >= nch))
        scat(nch - 2, (nch - 2) % 4).wait()
        scat(nch - 1, (nch - 1) % 4).wait()
        plsc.subcore_barrier()
        pltpu.sync_copy(acc_sh.at[pl.ds(s * stripe, stripe)],
                        out_hbm.at[c, pl.ds(s * stripe, stripe), pl.ds(0, F)])

    return k(scaled, ei3)


def _tc_matmul(x, W1, n_pad):
    n, F = x.shape[0], W1.shape[1]

    def k(x_ref, w_ref, out_ref):
        h = jnp.dot(x_ref[...], w_ref[...], preferred_element_type=jnp.float32)
        out_ref[0:n, :] = h
        out_ref[n:n_pad, :] = jnp.zeros((n_pad - n, F), jnp.float32)

    return pl.pallas_call(
        k,
        out_shape=jax.ShapeDtypeStruct((n_pad, F), jnp.float32),
    )(x, W1)


def _tc_first(deg0, deg1, h1):
    n_pad = h1.shape[0]

    def k(d0_ref, d1_ref, h_ref, out_scaled, out_dinv):
        deg = d0_ref[...] + d1_ref[...] + 1.0
        dv = lax.rsqrt(jnp.maximum(deg, 1.0))
        out_dinv[...] = dv
        out_scaled[...] = h_ref[...] * dv[:, None]

    return pl.pallas_call(
        k,
        out_shape=[
            jax.ShapeDtypeStruct(h1.shape, jnp.float32),
            jax.ShapeDtypeStruct((n_pad,), jnp.float32),
        ],
    )(deg0, deg1, h1)


def _tc_mid(p, sp, dinv, b, W):
    n_pad = sp.shape[0]
    Fo = W.shape[1]

    F = sp.shape[1]

    def k(p_ref, sp_ref, dinv_ref, b_ref, w_ref, out_ref):
        dcol = dinv_ref[...][:, None]
        combine = p_ref[0, :, 0:F] + p_ref[1, :, 0:F] - sp_ref[...]
        a = jnp.maximum(combine * dcol + b_ref[...], 0.0)
        h = jnp.dot(a, w_ref[...], preferred_element_type=jnp.float32)
        out_ref[...] = h * dcol

    return pl.pallas_call(
        k,
        out_shape=jax.ShapeDtypeStruct((n_pad, Fo), jnp.float32),
    )(p, sp, dinv, b, W)


def _tc_final(p, sp, dinv, b3, batch1, action,
              Wp1, bp1, Wp2a, Wp2b, bp2, Wp3, bp3, n_graphs):
    n_pad, F = sp.shape

    def k(p_ref, sp_ref, dinv_ref, b3_ref, batch_ref, act_ref,
          wp1_ref, bp1_ref, wp2a_ref, wp2b_ref, bp2_ref, wp3_ref, bp3_ref,
          out_ref):
        dcol = dinv_ref[...][:, None]
        combine = p_ref[0, :, 0:F] + p_ref[1, :, 0:F] - sp_ref[...]
        h3 = combine * dcol + b3_ref[...]
        bcol = batch_ref[...][:, None]
        gids = lax.broadcasted_iota(jnp.int32, (n_pad, n_graphs), 1)
        onehot = (bcol == gids).astype(jnp.float32)
        cdims = (((0,), (0,)), ((), ()))
        h3x = jnp.concatenate(
            [h3, jnp.ones((n_pad, 1), jnp.float32)], axis=1)
        sumsx = lax.dot_general(onehot, h3x, cdims,
                                preferred_element_type=jnp.float32)
        sums = sumsx[:, 0:h3.shape[1]]
        cnt = sumsx[:, h3.shape[1]:]
        pooled = sums / jnp.maximum(cnt, 1.0)
        fp = jnp.maximum(
            jnp.dot(pooled, wp1_ref[...], preferred_element_type=jnp.float32)
            + bp1_ref[...], 0.0)
        pol = (jnp.dot(fp, wp2a_ref[...], preferred_element_type=jnp.float32)
               + jnp.dot(act_ref[...], wp2b_ref[...],
                         preferred_element_type=jnp.float32)
               + bp2_ref[...])
        pol = jnp.maximum(pol, 0.0)
        out_ref[...] = (jnp.dot(pol, wp3_ref[...],
                                preferred_element_type=jnp.float32)
                        + bp3_ref[...])

    return pl.pallas_call(
        k,
        out_shape=jax.ShapeDtypeStruct((n_graphs, 1), jnp.float32),
    )(p, sp, dinv, b3, batch1, action,
      Wp1, bp1, Wp2a, Wp2b, bp2, Wp3, bp3)


def kernel(x, edge_index, batch, action,
           W1, b1, W2, b2, W3, b3, Wp1, bp1, Wp2, bp2, Wp3, bp3):
    n, _ = x.shape
    g = action.shape[0]
    e = edge_index.shape[1]
    assert e % CHUNK == 0
    rows_total = e // CHUNK
    base_rows = rows_total // NW
    rem = rows_total % NW

    n_pad = _round_up(n + N_SUBCORES, N_SUBCORES * 8 * 8)
    ei3 = edge_index.astype(jnp.int32).reshape(2, rows_total, CHUNK)

    batch1 = jnp.pad(batch.astype(jnp.int32), (0, n_pad - n),
                     constant_values=g)
    zeros = jnp.zeros((n_pad,), jnp.float32)

    deg0, deg1 = _sc_degree(ei3, zeros, n_pad, base_rows, rem, n)

    h1 = _tc_matmul(x, W1, n_pad)
    scaled1, dinv = _tc_first(deg0, deg1, h1)
    p1 = _sc_edge_pass(scaled1, ei3, n_pad, base_rows, rem, n)
    scaled2 = _tc_mid(p1, scaled1, dinv, b1[None, :], W2)
    p2 = _sc_edge_pass(scaled2, ei3, n_pad, base_rows, rem, n)
    scaled3 = _tc_mid(p2, scaled2, dinv, b2[None, :], W3)
    p3 = _sc_edge_pass(scaled3, ei3, n_pad, base_rows, rem, n)
    return _tc_final(p3, scaled3, dinv, b3[None, :], batch1, action,
                     Wp1, bp1[None, :], Wp2[:64], Wp2[64:],
                     bp2[None, :], Wp3, bp3[None, :], g)
